# baseline (speedup 1.0000x reference)
import functools

import jax
import jax.numpy as jnp
from jax import lax
from jax.experimental import pallas as pl
from jax.experimental.pallas import tpu as pltpu
from jax.experimental.pallas import tpu_sc as plsc
from jax._src.pallas import mpmd as _mpmd

N = 9000000
B = 1024
D = 256
GAMMA = 0.8
EPS = 1e-14
RHO = 8.0
BETA_U = 0.9
ETA = 0.01
GRAD_CLIP = 5.0
TAU_MIN = 0.001
TAU_MAX = 1.0
ALPHA = 0.5

NC = 2
NS = 16
NW = NC * NS
BPW = B // NW

_PREC = lax.Precision.HIGHEST


def _worker_base():
    wid = lax.axis_index("s") * NC + lax.axis_index("c")
    return wid * BPW


def _gather_body(ids_hbm, b0, b1, b2, b3, b4, b5, b6, b7, out_hbm,
                 idx_v, val_v, sem):
    base = _worker_base()
    pltpu.sync_copy(ids_hbm.at[pl.ds(base, BPW)], idx_v)
    bufs = (b0, b1, b2, b3, b4, b5, b6, b7)
    descs = []
    for k, buf in enumerate(bufs):
        descs.append(pltpu.async_copy(buf.at[idx_v], val_v.at[k], sem))
    for d in descs:
        d.wait()
    descs = []
    for k in range(8):
        descs.append(
            pltpu.async_copy(val_v.at[k], out_hbm.at[k, pl.ds(base, BPW)],
                             sem))
    for d in descs:
        d.wait()


@functools.lru_cache(maxsize=None)
def _get_sc_gather():
    mesh = plsc.VectorSubcoreMesh(core_axis_name="c", subcore_axis_name="s")
    return pl.kernel(
        _gather_body,
        out_type=jax.ShapeDtypeStruct((8, B), jnp.float32),
        mesh=mesh,
        scratch_types=[
            pltpu.VMEM((BPW,), jnp.int32),
            pltpu.VMEM((8, BPW), jnp.float32),
            pltpu.SemaphoreType.DMA,
        ],
    )


NCOPY = 3
CSPAN = 281248
NCHP = 8
CSZ = (35152, 35160)
CTAIL = N - NW * CSPAN


def _choff(ch):
    return (ch // 2) * (CSZ[0] + CSZ[1]) + (ch % 2) * CSZ[0]


def _sc_copy_body(i0, i1, i2, o0, o1, o2, buf_v0, buf_v1, tail_v,
                  sem_r0, sem_r1, sem_w0, sem_w1):
    del buf_v1, sem_r0, sem_r1, sem_w0, sem_w1
    wid = lax.axis_index("s") * NC + lax.axis_index("c")
    base = wid * CSPAN
    ins = (i0, i1, i2)
    outs = (o0, o1, o2)

    for c in range(NCHP * NCOPY):
        b, ch = divmod(c, NCHP)
        sz = CSZ[ch % 2]
        sl = pl.ds(base + _choff(ch), sz)
        pltpu.sync_copy(ins[b].at[sl], buf_v0.at[pl.ds(0, sz)])
        pltpu.sync_copy(buf_v0.at[pl.ds(0, sz)], outs[b].at[sl])

    toff = NW * CSPAN
    for b in range(NCOPY):
        pltpu.sync_copy(ins[b].at[pl.ds(toff, CTAIL)], tail_v)
        pltpu.sync_copy(tail_v, outs[b].at[pl.ds(toff, CTAIL)])


@functools.lru_cache(maxsize=None)
def _get_sc_copy():
    mesh = plsc.VectorSubcoreMesh(core_axis_name="c", subcore_axis_name="s")
    return pl.kernel(
        _sc_copy_body,
        out_type=[jax.ShapeDtypeStruct((N,), jnp.float32)] * NCOPY,
        mesh=mesh,
        scratch_types=[
            pltpu.VMEM((CSZ[1],), jnp.float32),
            pltpu.VMEM((CSZ[1],), jnp.float32),
            pltpu.VMEM((CTAIL,), jnp.float32),
            pltpu.SemaphoreType.DMA,
            pltpu.SemaphoreType.DMA,
            pltpu.SemaphoreType.DMA,
            pltpu.SemaphoreType.DMA,
        ],
    )


def _dense_body(zis_ref, zjs_ref, g_ref, idr_ref, v_ref, s_ref):
    z_i = zis_ref[:, :]
    z_j = zjs_ref[:, :]
    z_i = z_i / jnp.maximum(
        jnp.sqrt(jnp.sum(z_i * z_i, axis=1, keepdims=True)), 1e-12)
    z_j = z_j / jnp.maximum(
        jnp.sqrt(jnp.sum(z_j * z_j, axis=1, keepdims=True)), 1e-12)

    dn = (((1,), (1,)), ((), ()))
    dn0 = (((0,), (0,)), ((), ()))
    sim = lax.dot_general(z_i, z_j, dn, precision=lax.Precision.DEFAULT,
                          preferred_element_type=jnp.float32)
    simT = lax.dot_general(z_j, z_i, dn, precision=lax.Precision.DEFAULT,
                           preferred_element_type=jnp.float32)

    rc = lax.broadcasted_iota(jnp.int32, (B, B), 0)
    cc = lax.broadcasted_iota(jnp.int32, (B, B), 1)
    ondiag = rc == cc
    diag_col = jnp.sum(jnp.where(ondiag, sim, 0.0), axis=1, keepdims=True)

    i8 = lax.broadcasted_iota(jnp.int32, (8, 8), 0)
    j8 = lax.broadcasted_iota(jnp.int32, (8, 8), 1)
    eye8 = jnp.where(i8 == j8, 1.0, 0.0)
    g = lax.dot_general(g_ref[:, :], eye8, dn0, precision=_PREC,
                        preferred_element_type=jnp.float32)
    s_i_old = g[:, 0:1]
    s_t_old = g[:, 1:2]
    b_i_old = g[:, 2:3]
    b_t_old = g[:, 3:4]
    tau_i = g[:, 4:5]
    tau_t = g[:, 5:6]
    u_i_old = g[:, 6:7]
    u_t_old = g[:, 7:8]

    def side(mat, tau_col, b_old, s_old, u_old):
        inv_tau = 1.0 / tau_col
        dmax = jnp.max(mat, axis=1, keepdims=True) - diag_col
        b_new = jnp.maximum(dmax * inv_tau, b_old)
        off = inv_tau * diag_col + b_new
        e = jnp.where(ondiag, 0.0, jnp.exp(mat * inv_tau - off))
        se = jnp.sum(e, axis=1, keepdims=True)
        sem_ = jnp.sum(e * mat, axis=1, keepdims=True)
        sediffs = sem_ - diag_col * se
        sedt = sediffs * inv_tau
        s_new = (1.0 - GAMMA) * s_old * jnp.exp(b_old - b_new) + GAMMA * se
        s_c = jnp.maximum(s_new, EPS)
        loss_col = sediffs / s_c
        grad_tau = (jnp.log(s_c) + b_new + RHO
                    - sedt / s_c / (B - 1))
        u_new = (1.0 - BETA_U) * u_old + BETA_U * jnp.clip(
            grad_tau, -GRAD_CLIP, GRAD_CLIP)
        tau_new = jnp.clip(tau_col - ETA * u_new, TAU_MIN, TAU_MAX)
        return b_new, s_new, u_new, tau_new, loss_col

    b_i_new, s_i_new, u_i_new, tau_i_new, iloss = side(
        sim, tau_i, b_i_old, s_i_old, u_i_old)
    b_t_new, s_t_new, u_t_new, tau_t_new, tloss = side(
        simT, tau_t, b_t_old, s_t_old, u_t_old)

    total_loss = ALPHA * jnp.mean(iloss) + (1.0 - ALPHA) * jnp.mean(tloss)
    avg_i = jnp.mean(tau_i)
    avg_t = jnp.mean(tau_t)

    idr = idr_ref[:, :].astype(jnp.float32)
    idc = lax.dot_general(idr, jnp.ones((1, 1), jnp.float32), dn0,
                          precision=_PREC,
                          preferred_element_type=jnp.float32)
    eq = idc == idr
    last = jnp.max(jnp.where(eq, cc, -1), axis=1, keepdims=True)
    sel = jnp.where(cc == last, 1.0, 0.0)

    v = jnp.concatenate(
        [s_i_new, s_t_new, b_i_new, b_t_new,
         tau_i_new, tau_t_new, u_i_new, u_t_new], axis=1)
    v_ref[:, :] = lax.dot_general(v, sel, (((0,), (1,)), ((), ())),
                                  precision=_PREC,
                                  preferred_element_type=jnp.float32)

    lane = lax.broadcasted_iota(jnp.int32, (1, 128), 1)
    s_ref[:, :] = jnp.where(
        lane == 0, total_loss,
        jnp.where(lane == 1, avg_i, jnp.where(lane == 2, avg_t, 0.0)))


def _dense(zis, zjs, g8, ids_row):
    return pl.pallas_call(
        _dense_body,
        out_shape=[
            jax.ShapeDtypeStruct((8, B), jnp.float32),
            jax.ShapeDtypeStruct((1, 128), jnp.float32),
        ],
    )(zis, zjs, g8, ids_row)


def _scatter_body(b0, b1, b2, b3, b4, b5, b6, b7, ids_hbm, vals_hbm,
                  o0, o1, o2, o3, o4, o5, o6, o7, idx_v, val_v, sem):
    del b0, b1, b2, b3, b4, b5, b6, b7
    base = _worker_base()
    descs = [pltpu.async_copy(ids_hbm.at[pl.ds(base, BPW)], idx_v, sem)]
    for k in range(8):
        descs.append(
            pltpu.async_copy(vals_hbm.at[k, pl.ds(base, BPW)], val_v.at[k],
                             sem))
    for d in descs:
        d.wait()
    outs = (o0, o1, o2, o3, o4, o5, o6, o7)
    descs = []
    for k, out in enumerate(outs):
        descs.append(pltpu.async_copy(val_v.at[k], out.at[idx_v], sem))
    for d in descs:
        d.wait()


@functools.lru_cache(maxsize=None)
def _get_sc_scatter():
    mesh = plsc.VectorSubcoreMesh(core_axis_name="c", subcore_axis_name="s")
    return _mpmd._mpmd_map(
        [(mesh, _scatter_body)],
        out_types=[jax.ShapeDtypeStruct((N,), jnp.float32)] * 8,
        input_output_aliases={i: i for i in range(8)},
        scratch_types=[
            pltpu.VMEM((BPW,), jnp.int32),
            pltpu.VMEM((8, BPW), jnp.float32),
            pltpu.SemaphoreType.DMA,
        ],
    )


def kernel(zis, zjs, ids, s_I, s_T, b_I, b_T, tau_I, tau_T, u_I, u_T):
    bufs = (s_I, s_T, b_I, b_T, tau_I, tau_T, u_I, u_T)
    g8 = _get_sc_gather()(ids, *bufs)
    v8, s = _dense(zis, zjs, g8, ids.reshape(1, B))
    c0, c1, c2 = _get_sc_copy()(s_I, s_T, b_I)
    outs = _get_sc_scatter()(c0, c1, c2, b_T, tau_I, tau_T, u_I, u_T,
                             ids, v8)
    return (s[0, 0], s[0, 1], s[0, 2], *outs)


# speedup vs baseline: 1.0621x; 1.0621x over previous
import functools

import jax
import jax.numpy as jnp
from jax import lax
from jax.experimental import pallas as pl
from jax.experimental.pallas import tpu as pltpu
from jax.experimental.pallas import tpu_sc as plsc
from jax._src.pallas import mpmd as _mpmd

N = 9000000
B = 1024
D = 256
GAMMA = 0.8
EPS = 1e-14
RHO = 8.0
BETA_U = 0.9
ETA = 0.01
GRAD_CLIP = 5.0
TAU_MIN = 0.001
TAU_MAX = 1.0
ALPHA = 0.5

NC = 2
NS = 16
NW = NC * NS
BPW = B // NW

_PREC = lax.Precision.HIGHEST


def _worker_base():
    wid = lax.axis_index("s") * NC + lax.axis_index("c")
    return wid * BPW


def _gather_body(ids_hbm, b0, b1, b2, b3, b4, b5, b6, b7, out_hbm,
                 idx_v, val_v, sem):
    base = _worker_base()
    pltpu.sync_copy(ids_hbm.at[pl.ds(base, BPW)], idx_v)
    bufs = (b0, b1, b2, b3, b4, b5, b6, b7)
    descs = []
    for k, buf in enumerate(bufs):
        descs.append(pltpu.async_copy(buf.at[idx_v], val_v.at[k], sem))
    for d in descs:
        d.wait()
    descs = []
    for k in range(8):
        descs.append(
            pltpu.async_copy(val_v.at[k], out_hbm.at[k, pl.ds(base, BPW)],
                             sem))
    for d in descs:
        d.wait()


@functools.lru_cache(maxsize=None)
def _get_sc_gather():
    mesh = plsc.VectorSubcoreMesh(core_axis_name="c", subcore_axis_name="s")
    return pl.kernel(
        _gather_body,
        out_type=jax.ShapeDtypeStruct((8, B), jnp.float32),
        mesh=mesh,
        scratch_types=[
            pltpu.VMEM((BPW,), jnp.int32),
            pltpu.VMEM((8, BPW), jnp.float32),
            pltpu.SemaphoreType.DMA,
        ],
    )


def _dense_body(zis_ref, zjs_ref, g_ref, idr_ref, v_ref, s_ref):
    z_i = zis_ref[:, :]
    z_j = zjs_ref[:, :]
    z_i = z_i / jnp.maximum(
        jnp.sqrt(jnp.sum(z_i * z_i, axis=1, keepdims=True)), 1e-12)
    z_j = z_j / jnp.maximum(
        jnp.sqrt(jnp.sum(z_j * z_j, axis=1, keepdims=True)), 1e-12)

    dn = (((1,), (1,)), ((), ()))
    dn0 = (((0,), (0,)), ((), ()))
    sim = lax.dot_general(z_i, z_j, dn, precision=lax.Precision.DEFAULT,
                          preferred_element_type=jnp.float32)
    simT = lax.dot_general(z_j, z_i, dn, precision=lax.Precision.DEFAULT,
                           preferred_element_type=jnp.float32)

    rc = lax.broadcasted_iota(jnp.int32, (B, B), 0)
    cc = lax.broadcasted_iota(jnp.int32, (B, B), 1)
    ondiag = rc == cc
    diag_col = jnp.sum(jnp.where(ondiag, sim, 0.0), axis=1, keepdims=True)

    i8 = lax.broadcasted_iota(jnp.int32, (8, 8), 0)
    j8 = lax.broadcasted_iota(jnp.int32, (8, 8), 1)
    eye8 = jnp.where(i8 == j8, 1.0, 0.0)
    g = lax.dot_general(g_ref[:, :], eye8, dn0, precision=_PREC,
                        preferred_element_type=jnp.float32)
    s_i_old = g[:, 0:1]
    s_t_old = g[:, 1:2]
    b_i_old = g[:, 2:3]
    b_t_old = g[:, 3:4]
    tau_i = g[:, 4:5]
    tau_t = g[:, 5:6]
    u_i_old = g[:, 6:7]
    u_t_old = g[:, 7:8]

    def side(mat, tau_col, b_old, s_old, u_old):
        inv_tau = 1.0 / tau_col
        dmax = jnp.max(mat, axis=1, keepdims=True) - diag_col
        b_new = jnp.maximum(dmax * inv_tau, b_old)
        off = inv_tau * diag_col + b_new
        e = jnp.where(ondiag, 0.0, jnp.exp(mat * inv_tau - off))
        se = jnp.sum(e, axis=1, keepdims=True)
        sem_ = jnp.sum(e * mat, axis=1, keepdims=True)
        sediffs = sem_ - diag_col * se
        sedt = sediffs * inv_tau
        s_new = (1.0 - GAMMA) * s_old * jnp.exp(b_old - b_new) + GAMMA * se
        s_c = jnp.maximum(s_new, EPS)
        loss_col = sediffs / s_c
        grad_tau = (jnp.log(s_c) + b_new + RHO
                    - sedt / s_c / (B - 1))
        u_new = (1.0 - BETA_U) * u_old + BETA_U * jnp.clip(
            grad_tau, -GRAD_CLIP, GRAD_CLIP)
        tau_new = jnp.clip(tau_col - ETA * u_new, TAU_MIN, TAU_MAX)
        return b_new, s_new, u_new, tau_new, loss_col

    b_i_new, s_i_new, u_i_new, tau_i_new, iloss = side(
        sim, tau_i, b_i_old, s_i_old, u_i_old)
    b_t_new, s_t_new, u_t_new, tau_t_new, tloss = side(
        simT, tau_t, b_t_old, s_t_old, u_t_old)

    total_loss = ALPHA * jnp.mean(iloss) + (1.0 - ALPHA) * jnp.mean(tloss)
    avg_i = jnp.mean(tau_i)
    avg_t = jnp.mean(tau_t)

    idr = idr_ref[:, :].astype(jnp.float32)
    idc = lax.dot_general(idr, jnp.ones((1, 1), jnp.float32), dn0,
                          precision=_PREC,
                          preferred_element_type=jnp.float32)
    eq = idc == idr
    last = jnp.max(jnp.where(eq, cc, -1), axis=1, keepdims=True)
    sel = jnp.where(cc == last, 1.0, 0.0)

    v = jnp.concatenate(
        [s_i_new, s_t_new, b_i_new, b_t_new,
         tau_i_new, tau_t_new, u_i_new, u_t_new], axis=1)
    v_ref[:, :] = lax.dot_general(v, sel, (((0,), (1,)), ((), ())),
                                  precision=_PREC,
                                  preferred_element_type=jnp.float32)

    lane = lax.broadcasted_iota(jnp.int32, (1, 128), 1)
    s_ref[:, :] = jnp.where(
        lane == 0, total_loss,
        jnp.where(lane == 1, avg_i, jnp.where(lane == 2, avg_t, 0.0)))


def _dense(zis, zjs, g8, ids_row):
    return pl.pallas_call(
        _dense_body,
        out_shape=[
            jax.ShapeDtypeStruct((8, B), jnp.float32),
            jax.ShapeDtypeStruct((1, 128), jnp.float32),
        ],
    )(zis, zjs, g8, ids_row)


def _scatter_body(b0, b1, b2, b3, b4, b5, b6, b7, ids_hbm, vals_hbm,
                  o0, o1, o2, o3, o4, o5, o6, o7, idx_v, val_v, sem):
    del b0, b1, b2, b3, b4, b5, b6, b7
    base = _worker_base()
    descs = [pltpu.async_copy(ids_hbm.at[pl.ds(base, BPW)], idx_v, sem)]
    for k in range(8):
        descs.append(
            pltpu.async_copy(vals_hbm.at[k, pl.ds(base, BPW)], val_v.at[k],
                             sem))
    for d in descs:
        d.wait()
    outs = (o0, o1, o2, o3, o4, o5, o6, o7)
    descs = []
    for k, out in enumerate(outs):
        descs.append(pltpu.async_copy(val_v.at[k], out.at[idx_v], sem))
    for d in descs:
        d.wait()


@functools.lru_cache(maxsize=None)
def _get_sc_scatter():
    mesh = plsc.VectorSubcoreMesh(core_axis_name="c", subcore_axis_name="s")
    return _mpmd._mpmd_map(
        [(mesh, _scatter_body)],
        out_types=[jax.ShapeDtypeStruct((N,), jnp.float32)] * 8,
        input_output_aliases={i: i for i in range(8)},
        scratch_types=[
            pltpu.VMEM((BPW,), jnp.int32),
            pltpu.VMEM((8, BPW), jnp.float32),
            pltpu.SemaphoreType.DMA,
        ],
    )


def kernel(zis, zjs, ids, s_I, s_T, b_I, b_T, tau_I, tau_T, u_I, u_T):
    bufs = (s_I, s_T, b_I, b_T, tau_I, tau_T, u_I, u_T)
    g8 = _get_sc_gather()(ids, *bufs)
    v8, s = _dense(zis, zjs, g8, ids.reshape(1, B))
    outs = _get_sc_scatter()(*bufs, ids, v8)
    return (s[0, 0], s[0, 1], s[0, 2], *outs)
